# baseline (speedup 1.0000x reference)
import functools

import jax
import jax.numpy as jnp
from jax import lax
from jax.experimental import pallas as pl
from jax.experimental.pallas import tpu as pltpu
from jax.experimental.pallas import tpu_sc as plsc

H, W = 384, 1280
_EPS = 1e-6
_INF = 1e30
_N = 2000000
_C = 32
_NPAD = 2097152
_GRID = 16
_BLK = _NPAD // _GRID

_NW = 32
_PT = (H * W) // _NW
_CHUNK = 8192
_NCHUNK = _NPAD // _CHUNK
_FCH = 128


def _proj_body(p_ref, homt_ref, flat_ref, d_ref):
    pb = p_ref[...].astype(jnp.bfloat16)
    hb = homt_ref[...].astype(jnp.bfloat16)
    proj = jnp.dot(pb, hb, preferred_element_type=jnp.float32)
    ud = proj[0:1]
    vd = proj[1:2]
    d = proj[2:3]
    eps = jnp.float32(_EPS)
    safe = jnp.maximum(d, eps)
    px = jnp.floor(ud / safe).astype(jnp.int32)
    py = jnp.floor(vd / safe).astype(jnp.int32)
    valid = (d > eps) & (px >= 0) & (px < W) & (py >= 0) & (py < H)
    flat_ref[...] = jnp.where(valid, py * W + px, H * W).reshape(1, 1, _BLK)
    d_ref[...] = jnp.where(valid, d, jnp.float32(_INF)).reshape(1, 1, _BLK)


def _project(coordinates, projection_matrix):
    n = coordinates.shape[0]
    hom = jnp.concatenate([coordinates, jnp.ones((n, 1), coordinates.dtype)], axis=1)
    homt = jnp.pad(hom.T, ((0, 0), (0, _NPAD - n)))
    flat, d = pl.pallas_call(
        _proj_body,
        grid=(_GRID,),
        in_specs=[
            pl.BlockSpec((3, 4), lambda i: (0, 0)),
            pl.BlockSpec((4, _BLK), lambda i: (0, i)),
        ],
        out_specs=[
            pl.BlockSpec((1, 1, _BLK), lambda i: (i, 0, 0)),
            pl.BlockSpec((1, 1, _BLK), lambda i: (i, 0, 0)),
        ],
        out_shape=[
            jax.ShapeDtypeStruct((_GRID, 1, _BLK), jnp.int32),
            jax.ShapeDtypeStruct((_GRID, 1, _BLK), jnp.float32),
        ],
    )(projection_matrix, homt)
    return flat.reshape(_NPAD), d.reshape(_NPAD)


def _lane_gather(x, idx):
    return x.at[idx].get(mode="promise_in_bounds")


def _sc_zbuffer_body(flat_hbm, d_hbm, feats_hbm, feat_out, invd_out,
                     flat_v, d_v, flat_v2, d_v2, dbuf, ibuf, idx_v, rows4_v,
                     rows_v, sem, fsem1, dsem1, fsem2, dsem2):
    c = lax.axis_index("c")
    s = lax.axis_index("s")
    wid = c * 16 + s
    base = (wid * _PT).astype(jnp.int32)
    inf = jnp.float32(_INF)
    n_i = jnp.int32(_N)
    iota = lax.iota(jnp.int32, 16)

    def init_body(i, _):
        dbuf[pl.ds(i * 16, 16)] = jnp.full((16,), inf, jnp.float32)
        ibuf[pl.ds(i * 16, 16)] = jnp.full((16,), n_i, jnp.int32)
        return 0

    lax.fori_loop(0, _PT // 16, init_body, 0, unroll=False)

    def _issue(ci, fdst, ddst, fsem, dsem):
        pltpu.async_copy(flat_hbm.at[pl.ds(ci * _CHUNK, _CHUNK)], fdst, fsem)
        pltpu.async_copy(d_hbm.at[pl.ds(ci * _CHUNK, _CHUNK)], ddst, dsem)

    def _drain(fdst, ddst, fsem, dsem):
        pltpu.make_async_copy(flat_hbm.at[pl.ds(0, _CHUNK)], fdst, fsem).wait()
        pltpu.make_async_copy(d_hbm.at[pl.ds(0, _CHUNK)], ddst, dsem).wait()

    def chunk_pair_body(cp, _):
        ci0 = cp * 2
        _issue(ci0 + 1, flat_v2, d_v2, fsem2, dsem2)
        _drain(flat_v, d_v, fsem1, dsem1)
        _consume(ci0, flat_v, d_v)
        _issue(jnp.minimum(ci0 + 2, _NCHUNK - 1), flat_v, d_v, fsem1, dsem1)
        _drain(flat_v2, d_v2, fsem2, dsem2)
        _consume(ci0 + 1, flat_v2, d_v2)
        return 0

    def _consume(ci, fbuf, dbufv):
        def vec_body(k, _):
            f = fbuf[pl.ds(k * 16, 16)]
            lidx = f - base
            m = (lidx >= 0) & (lidx < _PT)
            cnt = plsc.all_reduce_population_count(m)

            @pl.when(cnt[0] == 1)
            def _single():
                dd = dbufv[pl.ds(k * 16, 16)]
                pidx = (ci * _CHUNK + k * 16) + iota
                lid1 = jnp.where(m, lidx, 0)
                cur_d = plsc.load_gather(dbuf, [lid1], mask=m)
                cur_i = plsc.load_gather(ibuf, [lid1], mask=m)
                take1 = m & ((dd < cur_d) | ((dd == cur_d) & (pidx < cur_i)))
                plsc.store_scatter(dbuf, [lid1], dd, mask=take1)
                plsc.store_scatter(ibuf, [lid1], pidx, mask=take1)

            @pl.when(cnt[0] > 1)
            def _process():
                dd = dbufv[pl.ds(k * 16, 16)]
                pidx = (ci * _CHUNK + k * 16) + iota
                ks, vs, ms = plsc.sort_key_val(lidx, iota, mask=m)
                ks = jnp.where(ms, ks, 0)
                msi = jnp.where(ms, 1, 0)
                dd_s = _lane_gather(dd, vs)
                pi_s = _lane_gather(pidx, vs)
                for off in (1, 2, 4, 8):
                    jc = jnp.maximum(iota - off, 0)
                    kp = _lane_gather(ks, jc)
                    dp = _lane_gather(dd_s, jc)
                    ip = _lane_gather(pi_s, jc)
                    same = (iota >= off) & (kp == ks)
                    take = same & ((dp < dd_s) | ((dp == dd_s) & (ip < pi_s)))
                    dd_s = jnp.where(take, dp, dd_s)
                    pi_s = jnp.where(take, ip, pi_s)
                jn = jnp.minimum(iota + 1, 15)
                kn = _lane_gather(ks, jn)
                msn = _lane_gather(msi, jn)
                is_end = ms & ((iota == 15) | (kn != ks) | (msn == 0))
                lid = jnp.where(is_end, ks, 0)
                cur_d = plsc.load_gather(dbuf, [lid], mask=is_end)
                cur_i = plsc.load_gather(ibuf, [lid], mask=is_end)
                take2 = is_end & ((dd_s < cur_d) | ((dd_s == cur_d) & (pi_s < cur_i)))
                plsc.store_scatter(dbuf, [lid], dd_s, mask=take2)
                plsc.store_scatter(ibuf, [lid], pi_s, mask=take2)

            return 0

        lax.fori_loop(0, _CHUNK // 16, vec_body, 0, unroll=4)

    _issue(0, flat_v, d_v, fsem1, dsem1)
    lax.fori_loop(0, _NCHUNK // 2, chunk_pair_body, 0, unroll=False)
    _drain(flat_v, d_v, fsem1, dsem1)

    eps = jnp.float32(_EPS)

    def invd_body(i, _):
        v = dbuf[pl.ds(i * 16, 16)]
        dbuf[pl.ds(i * 16, 16)] = jnp.where(
            v < inf, 1.0 / jnp.maximum(v, eps), jnp.float32(0.0))
        return 0

    lax.fori_loop(0, _PT // 16, invd_body, 0, unroll=False)
    pltpu.sync_copy(dbuf, invd_out.at[pl.ds(wid * _PT, _PT)])

    def fch_body(ch, _):
        row0 = ch * _FCH
        for g in range(_FCH // 16):
            v = ibuf[pl.ds(row0 + g * 16, 16)]
            w = jnp.minimum(v, n_i - 1)
            idx_v[pl.ds(g * 16, 16)] = lax.shift_right_logical(w, 2)
        pltpu.async_copy(feats_hbm.at[idx_v], rows4_v, sem).wait()
        for g in range(_FCH // 16):
            v = ibuf[pl.ds(row0 + g * 16, 16)]
            w = jnp.minimum(v, n_i - 1)
            off = (w & 3) * 32
            sc = jnp.where(v == n_i, jnp.float32(0.0), jnp.float32(1.0))
            for r in range(16):
                row = g * 16 + r
                rlane = jnp.full((16,), r, jnp.int32)
                rowsplat = jnp.full((16,), row, jnp.int32)
                offr = _lane_gather(off, rlane)
                scr = _lane_gather(sc, rlane)
                c0 = plsc.load_gather(rows4_v, [rowsplat, offr + iota])
                c1 = plsc.load_gather(rows4_v, [rowsplat, offr + iota + 16])
                rows_v[pl.ds(row * _C, 16)] = c0 * scr
                rows_v[pl.ds(row * _C + 16, 16)] = c1 * scr
        pltpu.sync_copy(
            rows_v, feat_out.at[pl.ds((wid * _PT + row0) * _C, _FCH * _C)])
        return 0

    lax.fori_loop(0, _PT // _FCH, fch_body, 0, unroll=False)


def _sc_zbuffer(flat, d, features):
    mesh = plsc.VectorSubcoreMesh(
        core_axis_name="c", subcore_axis_name="s", num_cores=2, num_subcores=16)
    kfn = pl.kernel(
        _sc_zbuffer_body,
        out_type=[
            jax.ShapeDtypeStruct((H * W * _C,), jnp.float32),
            jax.ShapeDtypeStruct((H * W,), jnp.float32),
        ],
        mesh=mesh,
        compiler_params=pltpu.CompilerParams(needs_layout_passes=False),
        scratch_types=[
            pltpu.VMEM((_CHUNK,), jnp.int32),
            pltpu.VMEM((_CHUNK,), jnp.float32),
            pltpu.VMEM((_CHUNK,), jnp.int32),
            pltpu.VMEM((_CHUNK,), jnp.float32),
            pltpu.VMEM((_PT,), jnp.float32),
            pltpu.VMEM((_PT,), jnp.int32),
            pltpu.VMEM((_FCH,), jnp.int32),
            pltpu.VMEM((_FCH, 128), jnp.float32),
            pltpu.VMEM((_FCH * _C,), jnp.float32),
            pltpu.SemaphoreType.DMA,
            pltpu.SemaphoreType.DMA,
            pltpu.SemaphoreType.DMA,
            pltpu.SemaphoreType.DMA,
            pltpu.SemaphoreType.DMA,
        ],
    )
    feats4 = features.reshape(_N // 4, 128)
    return kfn(flat, d, feats4)


def kernel(features, coordinates, projection_matrix):
    flat, d = _project(coordinates, projection_matrix)
    feat_out, invd = _sc_zbuffer(flat, d, features)
    return feat_out.reshape(H, W, _C), invd.reshape(H, W)


# speedup vs baseline: 1.0102x; 1.0102x over previous
import functools

import jax
import jax.numpy as jnp
from jax import lax
from jax.experimental import pallas as pl
from jax.experimental.pallas import tpu as pltpu
from jax.experimental.pallas import tpu_sc as plsc

H, W = 384, 1280
_EPS = 1e-6
_INF = 1e30
_N = 2000000
_C = 32
_NPAD = 2097152
_GRID = 16
_BLK = _NPAD // _GRID

_NW = 32
_PT = (H * W) // _NW
_CHUNK = 16384
_NCHUNK = _NPAD // _CHUNK
_FCH = 128


def _proj_body(p_ref, homt_ref, flat_ref, d_ref):
    pb = p_ref[...].astype(jnp.bfloat16)
    hb = homt_ref[...].astype(jnp.bfloat16)
    proj = jnp.dot(pb, hb, preferred_element_type=jnp.float32)
    ud = proj[0:1]
    vd = proj[1:2]
    d = proj[2:3]
    eps = jnp.float32(_EPS)
    safe = jnp.maximum(d, eps)
    px = jnp.floor(ud / safe).astype(jnp.int32)
    py = jnp.floor(vd / safe).astype(jnp.int32)
    valid = (d > eps) & (px >= 0) & (px < W) & (py >= 0) & (py < H)
    flat_ref[...] = jnp.where(valid, py * W + px, H * W).reshape(1, 1, _BLK)
    d_ref[...] = jnp.where(valid, d, jnp.float32(_INF)).reshape(1, 1, _BLK)


def _project(coordinates, projection_matrix):
    n = coordinates.shape[0]
    hom = jnp.concatenate([coordinates, jnp.ones((n, 1), coordinates.dtype)], axis=1)
    homt = jnp.pad(hom.T, ((0, 0), (0, _NPAD - n)))
    flat, d = pl.pallas_call(
        _proj_body,
        grid=(_GRID,),
        in_specs=[
            pl.BlockSpec((3, 4), lambda i: (0, 0)),
            pl.BlockSpec((4, _BLK), lambda i: (0, i)),
        ],
        out_specs=[
            pl.BlockSpec((1, 1, _BLK), lambda i: (i, 0, 0)),
            pl.BlockSpec((1, 1, _BLK), lambda i: (i, 0, 0)),
        ],
        out_shape=[
            jax.ShapeDtypeStruct((_GRID, 1, _BLK), jnp.int32),
            jax.ShapeDtypeStruct((_GRID, 1, _BLK), jnp.float32),
        ],
    )(projection_matrix, homt)
    return flat.reshape(_NPAD), d.reshape(_NPAD)


def _lane_gather(x, idx):
    return x.at[idx].get(mode="promise_in_bounds")


def _sc_zbuffer_body(flat_hbm, d_hbm, feats_hbm, feat_out, invd_out,
                     flat_v, d_v, flat_v2, d_v2, dbuf, ibuf, idx_v, rows4_v,
                     rows_v, sem, fsem1, dsem1, fsem2, dsem2):
    c = lax.axis_index("c")
    s = lax.axis_index("s")
    wid = c * 16 + s
    base = (wid * _PT).astype(jnp.int32)
    inf = jnp.float32(_INF)
    n_i = jnp.int32(_N)
    iota = lax.iota(jnp.int32, 16)

    def init_body(i, _):
        dbuf[pl.ds(i * 16, 16)] = jnp.full((16,), inf, jnp.float32)
        ibuf[pl.ds(i * 16, 16)] = jnp.full((16,), n_i, jnp.int32)
        return 0

    lax.fori_loop(0, _PT // 16, init_body, 0, unroll=False)

    def _issue(ci, fdst, ddst, fsem, dsem):
        pltpu.async_copy(flat_hbm.at[pl.ds(ci * _CHUNK, _CHUNK)], fdst, fsem)
        pltpu.async_copy(d_hbm.at[pl.ds(ci * _CHUNK, _CHUNK)], ddst, dsem)

    def _drain(fdst, ddst, fsem, dsem):
        pltpu.make_async_copy(flat_hbm.at[pl.ds(0, _CHUNK)], fdst, fsem).wait()
        pltpu.make_async_copy(d_hbm.at[pl.ds(0, _CHUNK)], ddst, dsem).wait()

    def chunk_pair_body(cp, _):
        ci0 = cp * 2
        _issue(ci0 + 1, flat_v2, d_v2, fsem2, dsem2)
        _drain(flat_v, d_v, fsem1, dsem1)
        _consume(ci0, flat_v, d_v)
        _issue(jnp.minimum(ci0 + 2, _NCHUNK - 1), flat_v, d_v, fsem1, dsem1)
        _drain(flat_v2, d_v2, fsem2, dsem2)
        _consume(ci0 + 1, flat_v2, d_v2)
        return 0

    def _consume(ci, fbuf, dbufv):
        def vec_body(k, _):
            f = fbuf[pl.ds(k * 16, 16)]
            lidx = f - base
            m = (lidx >= 0) & (lidx < _PT)
            cnt = plsc.all_reduce_population_count(m)

            @pl.when(cnt[0] == 1)
            def _single():
                dd = dbufv[pl.ds(k * 16, 16)]
                pidx = (ci * _CHUNK + k * 16) + iota
                lid1 = jnp.where(m, lidx, 0)
                cur_d = plsc.load_gather(dbuf, [lid1], mask=m)
                cur_i = plsc.load_gather(ibuf, [lid1], mask=m)
                take1 = m & ((dd < cur_d) | ((dd == cur_d) & (pidx < cur_i)))
                plsc.store_scatter(dbuf, [lid1], dd, mask=take1)
                plsc.store_scatter(ibuf, [lid1], pidx, mask=take1)

            @pl.when(cnt[0] > 1)
            def _process():
                dd = dbufv[pl.ds(k * 16, 16)]
                pidx = (ci * _CHUNK + k * 16) + iota
                ks, vs, ms = plsc.sort_key_val(lidx, iota, mask=m)
                ks = jnp.where(ms, ks, 0)
                msi = jnp.where(ms, 1, 0)
                dd_s = _lane_gather(dd, vs)
                pi_s = _lane_gather(pidx, vs)
                for off in (1, 2, 4, 8):
                    jc = jnp.maximum(iota - off, 0)
                    kp = _lane_gather(ks, jc)
                    dp = _lane_gather(dd_s, jc)
                    ip = _lane_gather(pi_s, jc)
                    same = (iota >= off) & (kp == ks)
                    take = same & ((dp < dd_s) | ((dp == dd_s) & (ip < pi_s)))
                    dd_s = jnp.where(take, dp, dd_s)
                    pi_s = jnp.where(take, ip, pi_s)
                jn = jnp.minimum(iota + 1, 15)
                kn = _lane_gather(ks, jn)
                msn = _lane_gather(msi, jn)
                is_end = ms & ((iota == 15) | (kn != ks) | (msn == 0))
                lid = jnp.where(is_end, ks, 0)
                cur_d = plsc.load_gather(dbuf, [lid], mask=is_end)
                cur_i = plsc.load_gather(ibuf, [lid], mask=is_end)
                take2 = is_end & ((dd_s < cur_d) | ((dd_s == cur_d) & (pi_s < cur_i)))
                plsc.store_scatter(dbuf, [lid], dd_s, mask=take2)
                plsc.store_scatter(ibuf, [lid], pi_s, mask=take2)

            return 0

        lax.fori_loop(0, _CHUNK // 16, vec_body, 0, unroll=8)

    _issue(0, flat_v, d_v, fsem1, dsem1)
    lax.fori_loop(0, _NCHUNK // 2, chunk_pair_body, 0, unroll=False)
    _drain(flat_v, d_v, fsem1, dsem1)

    eps = jnp.float32(_EPS)

    def invd_body(i, _):
        v = dbuf[pl.ds(i * 16, 16)]
        dbuf[pl.ds(i * 16, 16)] = jnp.where(
            v < inf, 1.0 / jnp.maximum(v, eps), jnp.float32(0.0))
        return 0

    lax.fori_loop(0, _PT // 16, invd_body, 0, unroll=False)
    pltpu.sync_copy(dbuf, invd_out.at[pl.ds(wid * _PT, _PT)])

    def fch_body(ch, _):
        row0 = ch * _FCH
        for g in range(_FCH // 16):
            v = ibuf[pl.ds(row0 + g * 16, 16)]
            w = jnp.minimum(v, n_i - 1)
            idx_v[pl.ds(g * 16, 16)] = lax.shift_right_logical(w, 2)
        pltpu.async_copy(feats_hbm.at[idx_v], rows4_v, sem).wait()
        for g in range(_FCH // 16):
            v = ibuf[pl.ds(row0 + g * 16, 16)]
            w = jnp.minimum(v, n_i - 1)
            off = (w & 3) * 32
            sc = jnp.where(v == n_i, jnp.float32(0.0), jnp.float32(1.0))
            for r in range(16):
                row = g * 16 + r
                rlane = jnp.full((16,), r, jnp.int32)
                rowsplat = jnp.full((16,), row, jnp.int32)
                offr = _lane_gather(off, rlane)
                scr = _lane_gather(sc, rlane)
                c0 = plsc.load_gather(rows4_v, [rowsplat, offr + iota])
                c1 = plsc.load_gather(rows4_v, [rowsplat, offr + iota + 16])
                rows_v[pl.ds(row * _C, 16)] = c0 * scr
                rows_v[pl.ds(row * _C + 16, 16)] = c1 * scr
        pltpu.sync_copy(
            rows_v, feat_out.at[pl.ds((wid * _PT + row0) * _C, _FCH * _C)])
        return 0

    lax.fori_loop(0, _PT // _FCH, fch_body, 0, unroll=False)


def _sc_zbuffer(flat, d, features):
    mesh = plsc.VectorSubcoreMesh(
        core_axis_name="c", subcore_axis_name="s", num_cores=2, num_subcores=16)
    kfn = pl.kernel(
        _sc_zbuffer_body,
        out_type=[
            jax.ShapeDtypeStruct((H * W * _C,), jnp.float32),
            jax.ShapeDtypeStruct((H * W,), jnp.float32),
        ],
        mesh=mesh,
        compiler_params=pltpu.CompilerParams(needs_layout_passes=False),
        scratch_types=[
            pltpu.VMEM((_CHUNK,), jnp.int32),
            pltpu.VMEM((_CHUNK,), jnp.float32),
            pltpu.VMEM((_CHUNK,), jnp.int32),
            pltpu.VMEM((_CHUNK,), jnp.float32),
            pltpu.VMEM((_PT,), jnp.float32),
            pltpu.VMEM((_PT,), jnp.int32),
            pltpu.VMEM((_FCH,), jnp.int32),
            pltpu.VMEM((_FCH, 128), jnp.float32),
            pltpu.VMEM((_FCH * _C,), jnp.float32),
            pltpu.SemaphoreType.DMA,
            pltpu.SemaphoreType.DMA,
            pltpu.SemaphoreType.DMA,
            pltpu.SemaphoreType.DMA,
            pltpu.SemaphoreType.DMA,
        ],
    )
    feats4 = features.reshape(_N // 4, 128)
    return kfn(flat, d, feats4)


def kernel(features, coordinates, projection_matrix):
    flat, d = _project(coordinates, projection_matrix)
    feat_out, invd = _sc_zbuffer(flat, d, features)
    return feat_out.reshape(H, W, _C), invd.reshape(H, W)


# speedup vs baseline: 1.0136x; 1.0033x over previous
import functools

import jax
import jax.numpy as jnp
from jax import lax
from jax.experimental import pallas as pl
from jax.experimental.pallas import tpu as pltpu
from jax.experimental.pallas import tpu_sc as plsc

H, W = 384, 1280
_EPS = 1e-6
_INF = 1e30
_N = 2000000
_C = 32
_NPAD = 2097152
_GRID = 16
_BLK = _NPAD // _GRID

_NW = 32
_PT = (H * W) // _NW
_CHUNK = 8192
_NCHUNK = _NPAD // _CHUNK
_FCH = 128


def _proj_body(p_ref, homt_ref, flat_ref, d_ref):
    pb = p_ref[...].astype(jnp.bfloat16)
    hb = homt_ref[...].astype(jnp.bfloat16)
    proj = jnp.dot(pb, hb, preferred_element_type=jnp.float32)
    ud = proj[0:1]
    vd = proj[1:2]
    d = proj[2:3]
    eps = jnp.float32(_EPS)
    safe = jnp.maximum(d, eps)
    px = jnp.floor(ud / safe).astype(jnp.int32)
    py = jnp.floor(vd / safe).astype(jnp.int32)
    valid = (d > eps) & (px >= 0) & (px < W) & (py >= 0) & (py < H)
    flat_ref[...] = jnp.where(valid, py * W + px, H * W).reshape(1, 1, _BLK)
    d_ref[...] = jnp.where(valid, d, jnp.float32(_INF)).reshape(1, 1, _BLK)


def _project(coordinates, projection_matrix):
    n = coordinates.shape[0]
    hom = jnp.concatenate([coordinates, jnp.ones((n, 1), coordinates.dtype)], axis=1)
    homt = jnp.pad(hom.T, ((0, 0), (0, _NPAD - n)))
    flat, d = pl.pallas_call(
        _proj_body,
        grid=(_GRID,),
        in_specs=[
            pl.BlockSpec((3, 4), lambda i: (0, 0)),
            pl.BlockSpec((4, _BLK), lambda i: (0, i)),
        ],
        out_specs=[
            pl.BlockSpec((1, 1, _BLK), lambda i: (i, 0, 0)),
            pl.BlockSpec((1, 1, _BLK), lambda i: (i, 0, 0)),
        ],
        out_shape=[
            jax.ShapeDtypeStruct((_GRID, 1, _BLK), jnp.int32),
            jax.ShapeDtypeStruct((_GRID, 1, _BLK), jnp.float32),
        ],
    )(projection_matrix, homt)
    return flat.reshape(_NPAD), d.reshape(_NPAD)


def _lane_gather(x, idx):
    return x.at[idx].get(mode="promise_in_bounds")


def _sc_zbuffer_body(flat_hbm, d_hbm, feats_hbm, feat_out, invd_out,
                     flat_v, d_v, flat_v2, d_v2, dbuf, ibuf, idx_v, idx_v2,
                     rows4_v, rows4_v2, rows_v, sem, fsem1, dsem1, fsem2,
                     dsem2):
    c = lax.axis_index("c")
    s = lax.axis_index("s")
    wid = c * 16 + s
    base = (wid * _PT).astype(jnp.int32)
    inf = jnp.float32(_INF)
    n_i = jnp.int32(_N)
    iota = lax.iota(jnp.int32, 16)

    def init_body(i, _):
        dbuf[pl.ds(i * 16, 16)] = jnp.full((16,), inf, jnp.float32)
        ibuf[pl.ds(i * 16, 16)] = jnp.full((16,), n_i, jnp.int32)
        return 0

    lax.fori_loop(0, _PT // 16, init_body, 0, unroll=False)

    def _issue(ci, fdst, ddst, fsem, dsem):
        pltpu.async_copy(flat_hbm.at[pl.ds(ci * _CHUNK, _CHUNK)], fdst, fsem)
        pltpu.async_copy(d_hbm.at[pl.ds(ci * _CHUNK, _CHUNK)], ddst, dsem)

    def _drain(fdst, ddst, fsem, dsem):
        pltpu.make_async_copy(flat_hbm.at[pl.ds(0, _CHUNK)], fdst, fsem).wait()
        pltpu.make_async_copy(d_hbm.at[pl.ds(0, _CHUNK)], ddst, dsem).wait()

    def chunk_pair_body(cp, _):
        ci0 = cp * 2
        _issue(ci0 + 1, flat_v2, d_v2, fsem2, dsem2)
        _drain(flat_v, d_v, fsem1, dsem1)
        _consume(ci0, flat_v, d_v)
        _issue(jnp.minimum(ci0 + 2, _NCHUNK - 1), flat_v, d_v, fsem1, dsem1)
        _drain(flat_v2, d_v2, fsem2, dsem2)
        _consume(ci0 + 1, flat_v2, d_v2)
        return 0

    def _consume(ci, fbuf, dbufv):
        def vec_body(k, _):
            f = fbuf[pl.ds(k * 16, 16)]
            lidx = f - base
            m = (lidx >= 0) & (lidx < _PT)
            cnt = plsc.all_reduce_population_count(m)

            @pl.when(cnt[0] == 1)
            def _single():
                dd = dbufv[pl.ds(k * 16, 16)]
                pidx = (ci * _CHUNK + k * 16) + iota
                lid1 = jnp.where(m, lidx, 0)
                cur_d = plsc.load_gather(dbuf, [lid1], mask=m)
                cur_i = plsc.load_gather(ibuf, [lid1], mask=m)
                take1 = m & ((dd < cur_d) | ((dd == cur_d) & (pidx < cur_i)))
                plsc.store_scatter(dbuf, [lid1], dd, mask=take1)
                plsc.store_scatter(ibuf, [lid1], pidx, mask=take1)

            @pl.when(cnt[0] > 1)
            def _process():
                dd = dbufv[pl.ds(k * 16, 16)]
                pidx = (ci * _CHUNK + k * 16) + iota
                ks, vs, ms = plsc.sort_key_val(lidx, iota, mask=m)
                ks = jnp.where(ms, ks, 0)
                msi = jnp.where(ms, 1, 0)
                dd_s = _lane_gather(dd, vs)
                pi_s = _lane_gather(pidx, vs)
                for off in (1, 2, 4, 8):
                    jc = jnp.maximum(iota - off, 0)
                    kp = _lane_gather(ks, jc)
                    dp = _lane_gather(dd_s, jc)
                    ip = _lane_gather(pi_s, jc)
                    same = (iota >= off) & (kp == ks)
                    take = same & ((dp < dd_s) | ((dp == dd_s) & (ip < pi_s)))
                    dd_s = jnp.where(take, dp, dd_s)
                    pi_s = jnp.where(take, ip, pi_s)
                jn = jnp.minimum(iota + 1, 15)
                kn = _lane_gather(ks, jn)
                msn = _lane_gather(msi, jn)
                is_end = ms & ((iota == 15) | (kn != ks) | (msn == 0))
                lid = jnp.where(is_end, ks, 0)
                cur_d = plsc.load_gather(dbuf, [lid], mask=is_end)
                cur_i = plsc.load_gather(ibuf, [lid], mask=is_end)
                take2 = is_end & ((dd_s < cur_d) | ((dd_s == cur_d) & (pi_s < cur_i)))
                plsc.store_scatter(dbuf, [lid], dd_s, mask=take2)
                plsc.store_scatter(ibuf, [lid], pi_s, mask=take2)

            return 0

        lax.fori_loop(0, _CHUNK // 16, vec_body, 0, unroll=8)

    _issue(0, flat_v, d_v, fsem1, dsem1)
    lax.fori_loop(0, _NCHUNK // 2, chunk_pair_body, 0, unroll=False)
    _drain(flat_v, d_v, fsem1, dsem1)

    eps = jnp.float32(_EPS)

    def invd_body(i, _):
        v = dbuf[pl.ds(i * 16, 16)]
        dbuf[pl.ds(i * 16, 16)] = jnp.where(
            v < inf, 1.0 / jnp.maximum(v, eps), jnp.float32(0.0))
        return 0

    lax.fori_loop(0, _PT // 16, invd_body, 0, unroll=False)
    pltpu.sync_copy(dbuf, invd_out.at[pl.ds(wid * _PT, _PT)])

    def _fissue(ch, ibuf_, dst, s_):
        row0 = ch * _FCH
        for g in range(_FCH // 16):
            v = ibuf_[pl.ds(row0 + g * 16, 16)]
            w = jnp.minimum(v, n_i - 1)
            idx_v[pl.ds(g * 16, 16)] = lax.shift_right_logical(w, 2)
        pltpu.async_copy(feats_hbm.at[idx_v], dst, s_)

    def _fissue2(ch, ibuf_, dst, s_):
        row0 = ch * _FCH
        for g in range(_FCH // 16):
            v = ibuf_[pl.ds(row0 + g * 16, 16)]
            w = jnp.minimum(v, n_i - 1)
            idx_v2[pl.ds(g * 16, 16)] = lax.shift_right_logical(w, 2)
        pltpu.async_copy(feats_hbm.at[idx_v2], dst, s_)

    def fch_pair_body(cp, _):
        ch0 = cp * 2
        _fissue2(ch0 + 1, ibuf, rows4_v2, fsem2)
        pltpu.make_async_copy(
            feats_hbm.at[idx_v], rows4_v, fsem1).wait()
        _fconsume(ch0, rows4_v)
        _fissue(jnp.minimum(ch0 + 2, _PT // _FCH - 1), ibuf, rows4_v, fsem1)
        pltpu.make_async_copy(
            feats_hbm.at[idx_v2], rows4_v2, fsem2).wait()
        _fconsume(ch0 + 1, rows4_v2)
        return 0

    def _fconsume(ch, r4):
        row0 = ch * _FCH
        for g in range(_FCH // 16):
            v = ibuf[pl.ds(row0 + g * 16, 16)]
            w = jnp.minimum(v, n_i - 1)
            off = (w & 3) * 32
            sc = jnp.where(v == n_i, jnp.float32(0.0), jnp.float32(1.0))
            for r in range(16):
                row = g * 16 + r
                rlane = jnp.full((16,), r, jnp.int32)
                rowsplat = jnp.full((16,), row, jnp.int32)
                offr = _lane_gather(off, rlane)
                scr = _lane_gather(sc, rlane)
                c0 = plsc.load_gather(r4, [rowsplat, offr + iota])
                c1 = plsc.load_gather(r4, [rowsplat, offr + iota + 16])
                rows_v[pl.ds(row * _C, 16)] = c0 * scr
                rows_v[pl.ds(row * _C + 16, 16)] = c1 * scr
        pltpu.sync_copy(
            rows_v, feat_out.at[pl.ds((wid * _PT + row0) * _C, _FCH * _C)])

    _fissue(0, ibuf, rows4_v, fsem1)
    lax.fori_loop(0, _PT // _FCH // 2, fch_pair_body, 0, unroll=False)
    pltpu.make_async_copy(feats_hbm.at[idx_v], rows4_v, fsem1).wait()


def _sc_zbuffer(flat, d, features):
    mesh = plsc.VectorSubcoreMesh(
        core_axis_name="c", subcore_axis_name="s", num_cores=2, num_subcores=16)
    kfn = pl.kernel(
        _sc_zbuffer_body,
        out_type=[
            jax.ShapeDtypeStruct((H * W * _C,), jnp.float32),
            jax.ShapeDtypeStruct((H * W,), jnp.float32),
        ],
        mesh=mesh,
        compiler_params=pltpu.CompilerParams(needs_layout_passes=False),
        scratch_types=[
            pltpu.VMEM((_CHUNK,), jnp.int32),
            pltpu.VMEM((_CHUNK,), jnp.float32),
            pltpu.VMEM((_CHUNK,), jnp.int32),
            pltpu.VMEM((_CHUNK,), jnp.float32),
            pltpu.VMEM((_PT,), jnp.float32),
            pltpu.VMEM((_PT,), jnp.int32),
            pltpu.VMEM((_FCH,), jnp.int32),
            pltpu.VMEM((_FCH,), jnp.int32),
            pltpu.VMEM((_FCH, 128), jnp.float32),
            pltpu.VMEM((_FCH, 128), jnp.float32),
            pltpu.VMEM((_FCH * _C,), jnp.float32),
            pltpu.SemaphoreType.DMA,
            pltpu.SemaphoreType.DMA,
            pltpu.SemaphoreType.DMA,
            pltpu.SemaphoreType.DMA,
            pltpu.SemaphoreType.DMA,
        ],
    )
    feats4 = features.reshape(_N // 4, 128)
    return kfn(flat, d, feats4)


def kernel(features, coordinates, projection_matrix):
    flat, d = _project(coordinates, projection_matrix)
    feat_out, invd = _sc_zbuffer(flat, d, features)
    return feat_out.reshape(H, W, _C), invd.reshape(H, W)
